# baseline (speedup 1.0000x reference)
import math
import jax
import jax.numpy as jnp
import numpy as np
from jax.experimental import pallas as pl

NUM_CHANNEL = 256
ANCHOR_RATIOS = (0.5, 1.0, 2.0)
ANCHOR_SIZES = (32, 64, 128, 256, 512)
ANCHOR_STRIDES = (4, 8, 16, 32, 64)
TEST_LONG_EDGE_SIZE = 512
MAX_SIZE = 512
PRE_NMS_TOPK = 6000
POST_NMS_TOPK = 1000
NMS_THRESH = 0.7
NA = 3

_SIZES = [(128, 128), (64, 64), (32, 32), (16, 16), (8, 8)]


def _gen_anchors(scale, ratios, shape, feature_stride, anchor_stride):
    scales, ratios = np.meshgrid(np.array([scale], dtype=np.float64), np.array(ratios, dtype=np.float64))
    scales = scales.flatten()
    ratios = ratios.flatten()
    size_ratios = feature_stride * feature_stride / ratios
    widths = np.round(np.sqrt(size_ratios))
    heights = np.round(widths * ratios)
    widths = widths * (scales / feature_stride)
    heights = heights * (scales / feature_stride)
    shifts_y = np.arange(0, shape[0], anchor_stride) * feature_stride + (feature_stride - 1) / 2
    shifts_x = np.arange(0, shape[1], anchor_stride) * feature_stride + (feature_stride - 1) / 2
    shifts_x, shifts_y = np.meshgrid(shifts_x, shifts_y)
    box_widths, box_centers_x = np.meshgrid(widths, shifts_x)
    box_heights, box_centers_y = np.meshgrid(heights, shifts_y)
    box_centers = np.stack([box_centers_y, box_centers_x], axis=2).reshape([-1, 2])
    box_sizes = np.stack([box_heights, box_widths], axis=2).reshape([-1, 2])
    boxes = np.concatenate([box_centers - 0.5 * (box_sizes - 1), box_centers + 0.5 * (box_sizes - 1)], axis=1)
    boxes[:, [2, 3]] += 1
    boxes = boxes[:, [1, 0, 3, 2]].astype(np.float32)
    return boxes


def _anchors_my_layout():
    out = []
    for i, (h, w) in enumerate(_SIZES):
        a = _gen_anchors(ANCHOR_SIZES[i], ANCHOR_RATIOS, (h, w), ANCHOR_STRIDES[i], 1)
        a = a.reshape(h * w, NA, 4).transpose(1, 0, 2).reshape(NA * h * w, 4)
        out.append(a)
    return np.concatenate(out, axis=0)

_ANCHORS_MY = _anchors_my_layout()
_N_TOT = _ANCHORS_MY.shape[0]


def kernel(image, feat0, feat1, feat2, feat3, feat4, conv0_w, conv0_b, cls_w, cls_b, box_w, box_b):
    feats = [feat0, feat1, feat2, feat3, feat4]
    wc = cls_w.reshape(NA, 256)
    wb = box_w.reshape(NA, 4, 256).transpose(1, 0, 2).reshape(12, 256)
    bc = cls_b
    bb = box_b.reshape(NA, 4).transpose(1, 0).reshape(12)
    w9 = conv0_w

    score_list = []
    dj_list = []
    for (h, w), x in zip(_SIZES, feats):
        xp = jnp.pad(x[0], ((0, 0), (1, 1), (1, 1)))
        acc = jnp.zeros((256, h * w), jnp.float32)
        for dy in range(3):
            for dx in range(3):
                xs = xp[:, dy:dy + h, dx:dx + w].reshape(256, h * w)
                acc = acc + jax.lax.dot_general(
                    w9[:, :, dy, dx], xs, (((1,), (0,)), ((), ())))
        hidden = jax.nn.relu(acc + conv0_b[:, None])
        sc = jax.lax.dot_general(wc, hidden, (((1,), (0,)), ((), ()))) + bc[:, None]
        dj = jax.lax.dot_general(wb, hidden, (((1,), (0,)), ((), ()))) + bb[:, None]
        score_list.append(sc.reshape(-1))
        dj_list.append(dj.reshape(4, -1))
    scores = jnp.concatenate(score_list)
    djs = jnp.concatenate(dj_list, axis=1)

    anc = jnp.asarray(_ANCHORS_MY)
    topk = PRE_NMS_TOPK
    top_scores, order = jax.lax.top_k(scores, topk)
    tx = djs[0][order]; ty = djs[1][order]; tw = djs[2][order]; th = djs[3][order]
    ax1 = anc[:, 0][order]; ay1 = anc[:, 1][order]
    ax2 = anc[:, 2][order]; ay2 = anc[:, 3][order]
    wa = ax2 - ax1; ha = ay2 - ay1
    xa = (ax2 + ax1) * 0.5; ya = (ay2 + ay1) * 0.5
    clip = jnp.float32(np.log(MAX_SIZE / 16.0))
    wbv = jnp.exp(jnp.minimum(tw, clip)) * wa
    hbv = jnp.exp(jnp.minimum(th, clip)) * ha
    xb = tx * wa + xa; yb = ty * ha + ya
    x1 = xb - wbv * 0.5; y1 = yb - hbv * 0.5
    x2 = xb + wbv * 0.5; y2 = yb + hbv * 0.5
    hh = float(image.shape[2]); ww = float(image.shape[3])
    x1 = x1.clip(0.0, hh); y1 = y1.clip(0.0, ww)
    x2 = x2.clip(0.0, hh); y2 = y2.clip(0.0, ww)
    boxes = jnp.stack([x1, y1, x2, y2], axis=1)

    n = boxes.shape[0]
    areas = (x2 - x1) * (y2 - y1)
    idxs = jnp.arange(n)

    def body(i, keep):
        xx1 = jnp.maximum(x1[i], x1)
        yy1 = jnp.maximum(y1[i], y1)
        xx2 = jnp.minimum(x2[i], x2)
        yy2 = jnp.minimum(y2[i], y2)
        iw = jnp.maximum(xx2 - xx1, 0.0)
        ih = jnp.maximum(yy2 - yy1, 0.0)
        inter = iw * ih
        iou = inter / (areas[i] + areas - inter + 1e-9)
        suppress = (iou > NMS_THRESH) & (idxs > i)
        return jnp.where(keep[i], keep & (~suppress), keep)

    keep = jax.lax.fori_loop(0, n, body, jnp.ones((n,), dtype=bool))
    masked = jnp.where(keep, top_scores, -jnp.inf)
    final_scores, idx = jax.lax.top_k(masked, POST_NMS_TOPK)
    proposals = boxes[idx]
    return proposals


# speedup vs baseline: 65.9435x; 65.9435x over previous
import functools
import math

import jax
import jax.numpy as jnp
import numpy as np
from jax import lax
from jax.experimental import pallas as pl
from jax.experimental.pallas import tpu as pltpu
from jax.experimental.pallas import tpu_sc as plsc

NA = 3
MAX_SIZE = 512
PRE_NMS_TOPK = 6000
POST_NMS_TOPK = 1000
NMS_THRESH = 0.7
_SIZES = [(128, 128), (64, 64), (32, 32), (16, 16), (8, 8)]
_N_TOT = sum(NA * h * w for h, w in _SIZES)
_N_PAD = 65536
_N_SEL = 6144
_CLIP = float(np.log(MAX_SIZE / 16.0))


def _gen_anchors(scale, ratios, shape, feature_stride, anchor_stride):
    scales, ratios = np.meshgrid(np.array([scale], dtype=np.float64), np.array(ratios, dtype=np.float64))
    scales = scales.flatten()
    ratios = ratios.flatten()
    size_ratios = feature_stride * feature_stride / ratios
    widths = np.round(np.sqrt(size_ratios))
    heights = np.round(widths * ratios)
    widths = widths * (scales / feature_stride)
    heights = heights * (scales / feature_stride)
    shifts_y = np.arange(0, shape[0], anchor_stride) * feature_stride + (feature_stride - 1) / 2
    shifts_x = np.arange(0, shape[1], anchor_stride) * feature_stride + (feature_stride - 1) / 2
    shifts_x, shifts_y = np.meshgrid(shifts_x, shifts_y)
    box_widths, box_centers_x = np.meshgrid(widths, shifts_x)
    box_heights, box_centers_y = np.meshgrid(heights, shifts_y)
    box_centers = np.stack([box_centers_y, box_centers_x], axis=2).reshape([-1, 2])
    box_sizes = np.stack([box_heights, box_widths], axis=2).reshape([-1, 2])
    boxes = np.concatenate([box_centers - 0.5 * (box_sizes - 1), box_centers + 0.5 * (box_sizes - 1)], axis=1)
    boxes[:, [2, 3]] += 1
    boxes = boxes[:, [1, 0, 3, 2]].astype(np.float32)
    return boxes


def _anchor_stats():
    rows = []
    for i, (h, w) in enumerate(_SIZES):
        a = _gen_anchors((32, 64, 128, 256, 512)[i], (0.5, 1.0, 2.0), (h, w), (4, 8, 16, 32, 64)[i], 1)
        a = a.reshape(h * w, NA, 4).transpose(1, 0, 2).reshape(NA * h * w, 4)
        rows.append(a)
    anc = np.concatenate(rows, axis=0)
    wa = anc[:, 2] - anc[:, 0]
    ha = anc[:, 3] - anc[:, 1]
    xa = (anc[:, 2] + anc[:, 0]) * 0.5
    ya = (anc[:, 3] + anc[:, 1]) * 0.5
    st = np.stack([wa, ha, xa, ya], axis=1).astype(np.float32)
    out = np.zeros((_N_PAD, 4), np.float32)
    out[:_N_TOT] = st
    return out

_ANC_STATS = _anchor_stats()


def _index_maps():
    r_of_m = np.zeros((_N_PAD,), np.int64)
    m_of_r = np.zeros((_N_PAD,), np.int64)
    base = 0
    for h, w in _SIZES:
        hw = h * w
        a = np.arange(NA)[:, None]
        pos = np.arange(hw)[None, :]
        m = base + a * hw + pos
        r = base + pos * 3 + a
        r_of_m[m.reshape(-1)] = r.reshape(-1)
        m_of_r[r.reshape(-1)] = m.reshape(-1)
        base += NA * hw
    pads = np.arange(_N_TOT, _N_PAD)
    r_of_m[pads] = pads
    m_of_r[pads] = pads
    return r_of_m.astype(np.float32), m_of_r.astype(np.int32)

_R_OF_M, _M_OF_R = _index_maps()


def _conv_head_body(H, W, xp_ref, w9_ref, b0_ref, wco_ref, bco_ref, out_ref):
    def row(r, _):
        x3 = xp_ref[pl.ds(r, 3)]
        acc = jnp.zeros((256, W), jnp.float32)
        for dy in range(3):
            xr = x3[dy]
            for dx in range(3):
                acc = acc + lax.dot_general(
                    w9_ref[dy * 3 + dx], xr[:, dx:dx + W],
                    (((1,), (0,)), ((), ())), preferred_element_type=jnp.float32)
        hidden = jnp.maximum(acc + b0_ref[:, 0][:, None], 0.0)
        o = lax.dot_general(wco_ref[...], hidden, (((1,), (0,)), ((), ())),
                            preferred_element_type=jnp.float32)
        out_ref[pl.ds(r, 1)] = (o + bco_ref[:, 0][:, None])[None]
        return 0

    lax.fori_loop(0, H, row, 0)


def _conv_head(xp_t, w9, b0, wco, bco, H, W):
    return pl.pallas_call(
        functools.partial(_conv_head_body, H, W),
        out_shape=jax.ShapeDtypeStruct((H, 16, W), jnp.float32),
    )(xp_t, w9, b0, wco, bco)


def _xor_partner(x, dist, axis, ilf):
    n = x.shape[axis]
    if axis == 0:
        up = jnp.concatenate([x[dist:, :], x[:dist, :]], axis=0)
        dn = jnp.concatenate([x[n - dist:, :], x[:n - dist, :]], axis=0)
    else:
        up = jnp.concatenate([x[:, dist:], x[:, :dist]], axis=1)
        dn = jnp.concatenate([x[:, n - dist:], x[:, :n - dist]], axis=1)
    return ilf * up + (1.0 - ilf) * dn


def _bitonic_stage(k, v, f, kk, j, colmajor, R, C):
    d = 1 << j
    if colmajor:
        axis, dist = (0, d) if d < R else (1, d // R)
    else:
        axis, dist = (1, d) if d < C else (0, d // C)
    bit_j = ((f >> j) & 1).astype(jnp.float32)
    ilf = 1.0 - bit_j
    pk = _xor_partner(k, dist, axis, ilf)
    pv = _xor_partner(v, dist, axis, ilf)
    tm = 1.0 - (((f >> j) ^ (f >> kk)) & 1).astype(jnp.float32)
    s = jnp.sign(k - pk)
    gt = jnp.maximum(s, 0.0)
    lt = jnp.maximum(-s, 0.0)
    tie = 1.0 - gt - lt
    vlt = jnp.maximum(jnp.sign(pv - v), 0.0)
    big = gt + tie * vlt
    sml = lt + tie * (1.0 - vlt)
    new_k = tm * jnp.maximum(k, pk) + (1.0 - tm) * jnp.minimum(k, pk)
    to = tm * sml + (1.0 - tm) * big
    new_v = to * pv + (1.0 - to) * v
    return new_k, new_v


def _bitonic_sort(k, v, f, nbits, colmajor, R, C):
    for kk in range(1, nbits + 1):
        for j in range(kk - 1, -1, -1):
            k, v = _bitonic_stage(k, v, f, kk, j, colmajor, R, C)
    return k, v


def _sort_phase_body(kks, keys_ref, v0_ref, ks_ref, vs_ref):
    R, C = 512, 128
    f = (lax.broadcasted_iota(jnp.int32, (R, C), 1) * R
         + lax.broadcasted_iota(jnp.int32, (R, C), 0))
    k, v = keys_ref[...], v0_ref[...]
    for kk in kks:
        for j in range(kk - 1, -1, -1):
            k, v = _bitonic_stage(k, v, f, kk, j, True, R, C)
    ks_ref[...] = k
    vs_ref[...] = v


def _sort_scores(keys2d, v02d):
    k, v = keys2d, v02d
    for kks in ((1, 2, 3, 4, 5, 6, 7, 8), (9, 10, 11), (12, 13), (14, 15), (16,)):
        k, v = pl.pallas_call(
            functools.partial(_sort_phase_body, kks),
            out_shape=(jax.ShapeDtypeStruct((512, 128), jnp.float32),
                       jax.ShapeDtypeStruct((512, 128), jnp.float32)),
        )(k, v)
    return k, v


def _gather_decode(order_i32, tcols):
    mesh = plsc.VectorSubcoreMesh(core_axis_name="c", subcore_axis_name="s")

    @functools.partial(
        pl.kernel, mesh=mesh,
        out_type=jax.ShapeDtypeStruct((32, 1536), jnp.float32),
        scratch_types=[
            pltpu.VMEM((192,), jnp.int32),
            pltpu.VMEM((192,), jnp.int32),
            pltpu.VMEM((192,), jnp.float32),
            pltpu.VMEM((192,), jnp.float32),
            pltpu.VMEM((192,), jnp.float32),
            pltpu.VMEM((192,), jnp.float32),
            pltpu.VMEM((192,), jnp.float32),
            pltpu.VMEM((192,), jnp.float32),
            pltpu.VMEM((192,), jnp.float32),
            pltpu.VMEM((192,), jnp.float32),
            pltpu.VMEM((1536,), jnp.float32),
            pltpu.SemaphoreType.DMA,
        ],
    )
    def k(order_hbm, mofr_hbm, t0, t1, t2, t3, t4, t5, t6, t7, out_hbm,
          idxr_v, idx_v, c0, c1, c2, c3, c4, c5, c6, c7, soa_v, sem):
        cols_v = (c0, c1, c2, c3, c4, c5, c6, c7)
        wid = lax.axis_index("s") * 2 + lax.axis_index("c")
        base = wid * 192
        pltpu.sync_copy(order_hbm.at[pl.ds(base, 192)], idxr_v)
        pltpu.async_copy(mofr_hbm.at[idxr_v], idx_v, sem).wait()
        handles = [pltpu.async_copy(t.at[idx_v], cv, sem)
                   for t, cv in zip((t0, t1, t2, t3, t4, t5, t6, t7), cols_v)]
        for h in handles:
            h.wait()
        zero16 = jnp.zeros((16,), jnp.float32)
        for g in range(12):
            sl = pl.ds(g * 16, 16)
            tx = cols_v[0][sl]; ty = cols_v[1][sl]
            tw = cols_v[2][sl]; th = cols_v[3][sl]
            wa = cols_v[4][sl]; ha = cols_v[5][sl]
            xa = cols_v[6][sl]; ya = cols_v[7][sl]
            wbv = jnp.exp(jnp.minimum(tw, _CLIP)) * wa
            hbv = jnp.exp(jnp.minimum(th, _CLIP)) * ha
            xb = tx * wa + xa
            yb = ty * ha + ya
            x1 = jnp.clip(xb - wbv * 0.5, 0.0, 512.0)
            y1 = jnp.clip(yb - hbv * 0.5, 0.0, 512.0)
            x2 = jnp.clip(xb + wbv * 0.5, 0.0, 512.0)
            y2 = jnp.clip(yb + hbv * 0.5, 0.0, 512.0)
            area = (x2 - x1) * (y2 - y1)
            o = g * 16
            soa_v[pl.ds(0 * 192 + o, 16)] = x1
            soa_v[pl.ds(1 * 192 + o, 16)] = y1
            soa_v[pl.ds(2 * 192 + o, 16)] = x2
            soa_v[pl.ds(3 * 192 + o, 16)] = y2
            soa_v[pl.ds(4 * 192 + o, 16)] = area
            soa_v[pl.ds(5 * 192 + o, 16)] = zero16
            soa_v[pl.ds(6 * 192 + o, 16)] = zero16
            soa_v[pl.ds(7 * 192 + o, 16)] = zero16
        pltpu.sync_copy(soa_v, out_hbm.at[wid])

    return k(order_i32, jnp.asarray(_M_OF_R), *tcols)


def _eqf(a, b):
    return 1.0 - jnp.minimum(jnp.abs(a - b), 1.0)


def _nms_body(soa_ref, sc_ref, aos_ref, out_ref):
    R, C = 48, 128
    fr = lax.broadcasted_iota(jnp.int32, (R, C), 0)
    fc = lax.broadcasted_iota(jnp.int32, (R, C), 1)
    fidx = fr * C + fc
    fidx_f = fidx.astype(jnp.float32)
    x1 = soa_ref[0]; y1 = soa_ref[1]; x2 = soa_ref[2]; y2 = soa_ref[3]
    areas = soa_ref[4]
    keep0 = jnp.minimum(jnp.maximum(jnp.float32(PRE_NMS_TOPK) - fidx_f, 0.0), 1.0)

    def cond(carry):
        i, cnt, _ = carry
        return (i < PRE_NMS_TOPK) & (cnt < jnp.float32(POST_NMS_TOPK))

    def body(carry):
        i, cnt, keep = carry
        i_f = i.astype(jnp.float32)
        oneh = _eqf(fidx_f, i_f)
        keep_i = jnp.sum(keep * oneh)
        bx1 = jnp.sum(x1 * oneh)
        by1 = jnp.sum(y1 * oneh)
        bx2 = jnp.sum(x2 * oneh)
        by2 = jnp.sum(y2 * oneh)
        bar = jnp.sum(areas * oneh)
        xx1 = jnp.maximum(bx1, x1)
        yy1 = jnp.maximum(by1, y1)
        xx2 = jnp.minimum(bx2, x2)
        yy2 = jnp.minimum(by2, y2)
        iw = jnp.maximum(xx2 - xx1, 0.0)
        ih = jnp.maximum(yy2 - yy1, 0.0)
        inter = iw * ih
        iou = inter / (bar + areas - inter + 1e-9)
        supf = (jnp.maximum(jnp.sign(iou - jnp.float32(NMS_THRESH)), 0.0)
                * jnp.maximum(jnp.sign(fidx_f - i_f), 0.0))
        keep = keep * (1.0 - keep_i * supf)
        return i + 1, cnt + keep_i, keep

    _, _, keep = lax.while_loop(cond, body, (jnp.int32(0), jnp.float32(0.0), keep0))

    sc = sc_ref[...]
    key48 = keep * sc - (1.0 - keep) * ((fidx_f + 2.0) * 1e9)
    f64i = (lax.broadcasted_iota(jnp.int32, (64, C), 0) * C
            + lax.broadcasted_iota(jnp.int32, (64, C), 1))
    f64f = f64i.astype(jnp.float32)
    key64 = jnp.concatenate(
        [key48, -(f64f[48:64] + 2.0) * 1e9], axis=0)
    _, v2 = _bitonic_sort(key64, f64f, f64i, 13, False, 64, C)
    perm = v2[0:8]

    p8 = (lax.broadcasted_iota(jnp.int32, (8, C), 0) * C
          + lax.broadcasted_iota(jnp.int32, (8, C), 1)).astype(jnp.float32)

    def gather_row(p, _):
        sel = _eqf(p8, p.astype(jnp.float32))
        idx = jnp.sum(perm * sel).astype(jnp.int32)
        out_ref[p] = aos_ref[idx]
        return 0

    lax.fori_loop(0, 1024, gather_row, 0)


def _nms_final(soa3d, sc48, aos):
    return pl.pallas_call(
        _nms_body,
        out_shape=jax.ShapeDtypeStruct((1024, 1, 8), jnp.float32),
    )(soa3d, sc48, aos)


def kernel(image, feat0, feat1, feat2, feat3, feat4, conv0_w, conv0_b, cls_w, cls_b, box_w, box_b):
    feats = [feat0, feat1, feat2, feat3, feat4]
    w9 = conv0_w.transpose(2, 3, 0, 1).reshape(9, 256, 256)
    wc = cls_w.reshape(NA, 256)
    wb = box_w.reshape(NA, 4, 256).transpose(1, 0, 2).reshape(12, 256)
    wco = jnp.concatenate([wc, wb, jnp.zeros((1, 256), jnp.float32)], axis=0)
    bco = jnp.concatenate(
        [cls_b, box_b.reshape(NA, 4).transpose(1, 0).reshape(12),
         jnp.zeros((1,), jnp.float32)])[:, None]
    b0 = conv0_b[:, None]

    outs = []
    for (h, w), x in zip(_SIZES, feats):
        xp_t = jnp.pad(x[0], ((0, 0), (1, 1), (1, 1))).transpose(1, 0, 2)
        o3 = _conv_head(xp_t, w9, b0, wco, bco, h, w)
        outs.append(o3.transpose(1, 0, 2).reshape(16, h * w))

    scores = jnp.concatenate([o[0:3].reshape(-1) for o in outs])
    djs = jnp.concatenate([o[3:15].reshape(4, -1) for o in outs], axis=1)

    keys = jnp.concatenate([scores, jnp.full((_N_PAD - _N_TOT,), -3.0e38, jnp.float32)])
    keys2d = keys.reshape(128, 512).T
    v02d = jnp.asarray(_R_OF_M).reshape(128, 512).T
    ks2d, vs2d = _sort_scores(keys2d, v02d)
    order = vs2d[:, :12].T.reshape(-1).astype(jnp.int32)
    top_scores = ks2d[:, :12].T.reshape(-1)

    djs_pad = jnp.pad(djs, ((0, 0), (0, _N_PAD - _N_TOT)))
    stats = jnp.asarray(_ANC_STATS)
    tcols = [djs_pad[j] for j in range(4)] + [stats[:, j] for j in range(4)]

    slabs = _gather_decode(order, tcols)
    soa = slabs.reshape(32, 8, 192).transpose(1, 0, 2).reshape(8, _N_SEL)
    soa3d = soa.reshape(8, 48, 128)
    aos = soa.T.reshape(_N_SEL, 1, 8)
    sc48 = top_scores.reshape(48, 128)

    out = _nms_final(soa3d, sc48, aos)
    return out.reshape(1024, 8)[:POST_NMS_TOPK, 0:4]
